# baseline (speedup 1.0000x reference)
import functools
import math

import numpy as np
import jax
from jax import lax
import jax.numpy as jnp
from jax.experimental import pallas as pl
from jax.experimental.pallas import tpu as pltpu
from jax.experimental.pallas import tpu_sc as plsc

S = 8
N = 10000
NPAD = 10240
H = 32
E = 160000
EPAD = 163840
WID = 128
W2 = 36
TN1 = 512
TN6 = 256
TE = 640
NC = 2
NS = 16
CH = 128


def _sin_pe(steps, d):
    pos = np.arange(steps)[:, None].astype(np.float32)
    div = np.exp(np.arange(0, d, 2).astype(np.float32) * (-math.log(10000.0) / d))
    pe = np.zeros((steps, d), np.float32)
    pe[:, 0::2] = np.sin(pos * div)
    pe[:, 1::2] = np.cos(pos * div)
    return jnp.asarray(pe)


def _ln(x, g, b, eps=1e-5):
    mu = jnp.mean(x, axis=-1, keepdims=True)
    var = jnp.mean((x - mu) ** 2, axis=-1, keepdims=True)
    return (x - mu) * jax.lax.rsqrt(var + eps) * g + b


def _mm(x, w):
    sh = x.shape
    x2 = x.reshape(-1, sh[-1])
    y = jnp.dot(x2, w, preferred_element_type=jnp.float32)
    return y.reshape(sh[:-1] + (w.shape[-1],))


def _silu(x):
    return x * jax.nn.sigmoid(x)


def _prep_body(x_ref, mask_ref, u_ref, emb_ref, pe_ref,
               wu_ref, bu_ref, w1_ref, b1_ref, w2_ref, b2_ref,
               wm1_ref, bm1_ref, wm2_ref, bm2_ref, g_ref, bn_ref,
               h_ref):
    x = x_ref[...]
    mask = mask_ref[...]
    u = u_ref[...]
    xw = x * mask
    qu = jnp.dot(u, wu_ref[...], preferred_element_type=jnp.float32) + bu_ref[...]
    q = qu[:, None, :] + emb_ref[...][None, :, :]
    q = jnp.where(q > 0, q, 0.01 * q)
    q = jnp.maximum(_mm(q, wm1_ref[...]) + bm1_ref[...], 0.0)
    q = jnp.maximum(_mm(q, wm2_ref[...]) + bm2_ref[...], 0.0)
    q = q + pe_ref[...][:, None, :]
    h = jnp.maximum(xw * w1_ref[...] + b1_ref[...], 0.0)
    h = jnp.maximum(_mm(h, w2_ref[...]) + b2_ref[...], 0.0)
    h = jnp.where(mask > 0.5, h + q, q)
    h_ref[...] = _ln(h, g_ref[...], bn_ref[...])


def _full(shape):
    nd = len(shape)
    return pl.BlockSpec(shape, lambda i: (0,) * nd)


def _prep(xp, maskp, u, embp, pe, p):
    grid = NPAD // TN1
    ns = lambda w: pl.BlockSpec((S, TN1, w), lambda i: (0, i, 0))
    ins = [
        ns(1), ns(1), _full((S, 8)),
        pl.BlockSpec((TN1, H), lambda i: (i, 0)), _full((S, H)),
        _full((8, H)), _full((1, H)), _full((1, H)), _full((1, H)),
        _full((H, H)), _full((1, H)), _full((H, H)), _full((1, H)),
        _full((H, H)), _full((1, H)), _full((1, H)), _full((1, H)),
    ]
    args = (
        xp, maskp, u, embp, pe,
        p["u_lin"]["W"], p["u_lin"]["b"].reshape(1, H),
        p["h_enc1"]["W"].reshape(1, H), p["h_enc1"]["b"].reshape(1, H),
        p["h_enc2"]["W"], p["h_enc2"]["b"].reshape(1, H),
        p["u_mlp1"]["W"], p["u_mlp1"]["b"].reshape(1, H),
        p["u_mlp2"]["W"], p["u_mlp2"]["b"].reshape(1, H),
        p["h_norm"]["g"].reshape(1, H), p["h_norm"]["b"].reshape(1, H),
    )
    return pl.pallas_call(
        _prep_body,
        grid=(grid,),
        in_specs=ins,
        out_specs=pl.BlockSpec((S, TN1, H), lambda i: (0, i, 0)),
        out_shape=jax.ShapeDtypeStruct((S, NPAD, H), jnp.float32),
    )(*args)


def _pre_body(use_emb, use_mask,
              h_ref, x_ref, mask_ref, vemb_ref, memb_ref,
              wxs_ref, bxs_ref, ws_ref, bs_ref, wt_ref, bt_ref, gb_ref,
              hp_ref, zs_ref, zt_ref):
    h = h_ref[...]
    x = x_ref[...]
    mask = mask_ref[...]
    if use_emb:
        h = h + jnp.where(mask > 0.5, vemb_ref[...][None], memb_ref[...][None])
    h = h + (x * wxs_ref[...] + bxs_ref[...]) * mask
    hp_ref[...] = h
    zs = _mm(h, ws_ref[...]) + bs_ref[...]
    zt = _mm(h, wt_ref[...])
    gb = gb_ref[...][None]
    if use_mask:
        mb = jnp.where(mask > 0.5, 0.0, -1e9) + gb
    else:
        mb = jnp.zeros_like(mask) + gb
    pad = jnp.zeros(zs.shape[:-1] + (WID - H - 1,), jnp.float32)
    zs_ref[...] = jnp.concatenate([zs, mb, pad], axis=-1)
    zt_ref[...] = jnp.concatenate([zt, jnp.zeros_like(mb), pad], axis=-1)


def _pre(hs, xp, maskp, p, lp, layer_idx):
    grid = NPAD // TN1
    ns = lambda w: pl.BlockSpec((S, TN1, w), lambda i: (0, i, 0))
    es = pl.BlockSpec((TN1, H), lambda i: (i, 0))
    bias_total = (lp["sp_src"]["b"] + lp["sp_tgt"]["b"] + lp["sp_bias"]).reshape(1, H)
    args = (
        hs, xp, maskp, p["valid_emb_p"], p["mask_emb_p"],
        lp["x_skip"]["W"].reshape(1, H), lp["x_skip"]["b"].reshape(1, H),
        lp["sp_src"]["W"], bias_total, lp["sp_tgt"]["W"],
        jnp.zeros((1, H), jnp.float32),
        lp["sp_gate"]["b"].reshape(1, 1),
    )
    ins = [ns(H), ns(1), ns(1), es, es,
           _full((1, H)), _full((1, H)), _full((H, H)), _full((1, H)),
           _full((H, H)), _full((1, H)), _full((1, 1))]
    body = functools.partial(_pre_body, layer_idx == 1, layer_idx == 0)
    return pl.pallas_call(
        body,
        grid=(grid,),
        in_specs=ins,
        out_specs=[ns(H), ns(WID), ns(WID)],
        out_shape=[
            jax.ShapeDtypeStruct((S, NPAD, H), jnp.float32),
            jax.ShapeDtypeStruct((S, NPAD, WID), jnp.float32),
            jax.ShapeDtypeStruct((S, NPAD, WID), jnp.float32),
        ],
    )(*args)


def _edge_body(a1_ref, a2_ref, g_ref, wm_ref, p_ref):
    a = a1_ref[...] + a2_ref[...]
    feats = a[:, :, :H]
    m = _silu(feats)
    score = jnp.sum(m * g_ref[...][0, None, None, :], axis=-1, keepdims=True)
    score = score + a[:, :, H:H + 1]
    w = jnp.exp(score)
    msg = _silu(_mm(m, wm_ref[...]))
    pad = jnp.zeros((a.shape[0], a.shape[1], W2 - H - 1), jnp.float32)
    p_ref[...] = jnp.concatenate([msg * w, w, pad], axis=-1)


def _edge(A1, A2, lp):
    g = lp["sp_gate"]["W"].reshape(1, H)
    sg = A1.shape[0]
    grid = EPAD // TE
    es = pl.BlockSpec((sg, TE, WID), lambda i: (0, i, 0))
    os = pl.BlockSpec((sg, TE, W2), lambda i: (0, i, 0))
    return pl.pallas_call(
        _edge_body,
        grid=(grid,),
        in_specs=[es, es, _full((1, H)), _full((H, H))],
        out_specs=os,
        out_shape=jax.ShapeDtypeStruct((sg, EPAD, W2), jnp.float32),
    )(A1, A2, g, lp["sp_msg"]["W"])


def _sc_gather(zs_flat, zt_flat, sidx, tidx):
    nrows = sidx.shape[0]
    rows_per = nrows // (NC * NS)
    nchunk = rows_per // CH
    mesh = plsc.VectorSubcoreMesh(core_axis_name="c", subcore_axis_name="s")

    @functools.partial(
        pl.kernel, mesh=mesh,
        out_type=[
            jax.ShapeDtypeStruct((nrows, WID), jnp.float32),
            jax.ShapeDtypeStruct((nrows, WID), jnp.float32),
        ],
        scratch_types=[
            pltpu.VMEM((CH,), jnp.int32),
            pltpu.VMEM((CH,), jnp.int32),
            pltpu.VMEM((CH, WID), jnp.float32),
            pltpu.VMEM((CH, WID), jnp.float32),
            pltpu.SemaphoreType.DMA,
            pltpu.SemaphoreType.DMA,
        ],
    )
    def k(zs_hbm, zt_hbm, si_hbm, ti_hbm, a1_hbm, a2_hbm,
          idx1, idx2, r1, r2, sem1, sem2):
        cid = lax.axis_index("c")
        sid = lax.axis_index("s")
        wid = sid * NC + cid
        base_t = wid * rows_per

        def body(i, carry):
            base = base_t + i * CH
            pltpu.sync_copy(si_hbm.at[pl.ds(base, CH)], idx1)
            pltpu.sync_copy(ti_hbm.at[pl.ds(base, CH)], idx2)
            cp1 = pltpu.async_copy(zs_hbm.at[idx1], r1, sem1)
            cp2 = pltpu.async_copy(zt_hbm.at[idx2], r2, sem2)
            cp1.wait()
            cp2.wait()
            pltpu.sync_copy(r1, a1_hbm.at[pl.ds(base, CH)])
            pltpu.sync_copy(r2, a2_hbm.at[pl.ds(base, CH)])
            return carry

        lax.fori_loop(0, nchunk, body, 0)

    return k(zs_flat, zt_flat, sidx, tidx)


def _sc_scatter(P_flat, tgt_pad, zrows):
    half = EPAD // NC
    per_tile = half // NS
    nchunk = per_tile // CH
    srows = NPAD // NS
    nst = P_flat.shape[0] // EPAD
    mesh = plsc.VectorSubcoreMesh(core_axis_name="c", subcore_axis_name="s")

    @functools.partial(
        pl.kernel, mesh=mesh,
        out_type=jax.ShapeDtypeStruct((NC, nst, NPAD, W2), jnp.float32),
        scratch_types=[
            pltpu.VMEM((CH,), jnp.int32),
            pltpu.VMEM((CH, W2), jnp.float32),
            pltpu.VMEM_SHARED((NPAD, W2), jnp.float32),
        ],
    )
    def k(p_hbm, tgt_hbm, z_hbm, out_hbm, idx1, rows, table):
        cid = lax.axis_index("c")
        sid = lax.axis_index("s")
        myslice = pl.ds(sid * srows, srows)
        pltpu.sync_copy(z_hbm, table.at[myslice])
        plsc.subcore_barrier()

        def step_body(s, carry):
            ebase0 = cid * half + sid * per_tile

            def chunk(i, c2):
                eb = ebase0 + i * CH
                pltpu.sync_copy(tgt_hbm.at[pl.ds(eb, CH)], idx1)
                pltpu.sync_copy(p_hbm.at[pl.ds(s * EPAD + eb, CH)], rows)
                pltpu.sync_copy(rows, table.at[idx1], add=True)
                return c2

            lax.fori_loop(0, nchunk, chunk, 0)
            plsc.subcore_barrier()
            pltpu.sync_copy(table.at[myslice], out_hbm.at[cid, s, myslice])
            pltpu.sync_copy(z_hbm, table.at[myslice])
            plsc.subcore_barrier()
            return carry

        lax.fori_loop(0, nst, step_body, 0)

    return k(P_flat, tgt_pad, zrows)


def _post_body(hp_ref, p1_ref, p2_ref, mask_ref,
               wts_ref, bts_ref, wtt_ref, wtm_ref, btm_ref, gt_ref, gtb_ref,
               wsk_ref, bsk_ref, gn_ref, bn_ref,
               w1_ref, b1_ref, w2_ref, b2_ref, w3_ref, b3_ref,
               h_ref, imp_ref):
    hp = hp_ref[...]
    mask = mask_ref[...]
    psum = p1_ref[...] + p2_ref[...]
    out_sp = psum[:, :, :H] / (psum[:, :, H:H + 1] + 1e-16)

    zs = _mm(hp, wts_ref[...]) + bts_ref[...]
    zt = _mm(hp, wtt_ref[...])
    pair = _silu(zs[None, :, :, :] + zt[:, None, :, :])
    score = jnp.sum(pair * gt_ref[...][0, None, None, None, :], axis=-1)
    score = score + gtb_ref[...][:, :, None]
    tmask = (mask[:, :, 0] > 0.5)[None, :, :]
    score = jnp.where(tmask, score, -1e9)
    smax = jnp.max(score, axis=1, keepdims=True)
    ex = jnp.exp(score - smax)
    alpha = ex / jnp.sum(ex, axis=1, keepdims=True)
    msg = _silu(_mm(pair, wtm_ref[...]))
    out_tm = jnp.sum(alpha[..., None] * msg, axis=1)

    hn = out_sp + out_tm + _mm(hp, wsk_ref[...]) + bsk_ref[...]
    hn = _ln(hn, gn_ref[...], bn_ref[...])
    h_ref[...] = hn
    r = jnp.maximum(_mm(hn, w1_ref[...]) + b1_ref[...], 0.0)
    r = jnp.maximum(_mm(r, w2_ref[...]) + b2_ref[...], 0.0)
    imp = jnp.sum(r * w3_ref[...][0, None, None, :], axis=-1, keepdims=True)
    imp_ref[...] = imp + b3_ref[...][None]


def _post(hp, part1, part2, maskp, lp, gate_b):
    grid = NPAD // TN6
    ns = lambda w: pl.BlockSpec((S, TN6, w), lambda i: (0, i, 0))
    bts = (lp["tm_src"]["b"] + lp["tm_tgt"]["b"] + lp["tm_bias"]).reshape(1, H)
    args = (
        hp, part1, part2, maskp,
        lp["tm_src"]["W"], bts, lp["tm_tgt"]["W"],
        lp["tm_msg"]["W"], lp["tm_msg"]["b"].reshape(1, H),
        lp["tm_gate"]["W"].reshape(1, H), lp["tm_gate"]["b"].reshape(1, 1),
        lp["skip"]["W"], lp["skip"]["b"].reshape(1, H),
        lp["norm"]["g"].reshape(1, H), lp["norm"]["b"].reshape(1, H),
        lp["ro1"]["W"], lp["ro1"]["b"].reshape(1, H),
        lp["ro2"]["W"], lp["ro2"]["b"].reshape(1, H),
        lp["ro3"]["W"].reshape(1, H), lp["ro3"]["b"].reshape(1, 1),
    )
    ins = [ns(H), ns(W2), ns(W2), ns(1),
           _full((H, H)), _full((1, H)), _full((H, H)),
           _full((H, H)), _full((1, H)), _full((1, H)), _full((1, 1)),
           _full((H, H)), _full((1, H)), _full((1, H)), _full((1, H)),
           _full((H, H)), _full((1, H)), _full((H, H)), _full((1, H)),
           _full((1, H)), _full((1, 1))]
    return pl.pallas_call(
        _post_body,
        grid=(grid,),
        in_specs=ins,
        out_specs=[ns(H), ns(1)],
        out_shape=[
            jax.ShapeDtypeStruct((S, NPAD, H), jnp.float32),
            jax.ShapeDtypeStruct((S, NPAD, 1), jnp.float32),
        ],
    )(*args)


def kernel(x, u, mask, edge_index, params):
    x3 = x[0]
    mask3 = mask[0]
    u2 = u[0]
    src = edge_index[0]
    tgt = edge_index[1]

    padn = ((0, 0), (0, NPAD - N), (0, 0))
    xp = jnp.pad(x3, padn)
    maskp = jnp.pad(mask3, padn)
    embp = jnp.pad(params["u_node_emb"], ((0, NPAD - N), (0, 0)))
    pe = _sin_pe(S, H)

    p = dict(params)
    p["valid_emb_p"] = jnp.pad(params["valid_emb"], ((0, NPAD - N), (0, 0)))
    p["mask_emb_p"] = jnp.pad(params["mask_emb"], ((0, NPAD - N), (0, 0)))

    h = _prep(xp, maskp, u2, embp, pe, p)

    src_pad = jnp.concatenate([src, jnp.zeros((EPAD - E,), jnp.int32)])
    tgt_pad = jnp.concatenate(
        [tgt, jnp.full((EPAD - E,), NPAD - 1, jnp.int32)])
    steps_off = (jnp.arange(S, dtype=jnp.int32) * NPAD)[:, None]
    sidx = (steps_off + src_pad[None, :]).reshape(-1)
    tidx = (steps_off + tgt_pad[None, :]).reshape(-1)
    zrows = jnp.zeros((NPAD // NS, W2), jnp.float32)

    NG = 4
    G = S // NG
    sidx_g = [sidx.reshape(S, EPAD)[g * G:(g + 1) * G].reshape(-1)
              for g in range(NG)]
    tidx_g = [tidx.reshape(S, EPAD)[g * G:(g + 1) * G].reshape(-1)
              for g in range(NG)]

    imps = []
    for l, lp in enumerate(params["layers"]):
        hp, zs_ext, zt_ext = _pre(h, xp, maskp, p, lp, l)
        zs_flat = zs_ext.reshape(S * NPAD, WID)
        zt_flat = zt_ext.reshape(S * NPAD, WID)
        parts_gs = []
        for g in range(NG):
            A1, A2 = _sc_gather(zs_flat, zt_flat, sidx_g[g], tidx_g[g])
            P = _edge(A1.reshape(G, EPAD, WID), A2.reshape(G, EPAD, WID), lp)
            parts_gs.append(_sc_scatter(P.reshape(G * EPAD, W2), tgt_pad, zrows))
        parts = jnp.concatenate(parts_gs, axis=1)
        h, imp = _post(hp, parts[0], parts[1], maskp, lp, None)
        imps.append(imp[:, :N, :][None])

    return (imps[1], imps[0])


# speedup vs baseline: 1.2568x; 1.2568x over previous
import functools
import math

import numpy as np
import jax
from jax import lax
import jax.numpy as jnp
from jax.experimental import pallas as pl
from jax.experimental.pallas import tpu as pltpu
from jax.experimental.pallas import tpu_sc as plsc

S = 8
N = 10000
NPAD = 10240
H = 32
E = 160000
EPAD = 163840
WID = 128
W2 = 36
TN1 = 512
TN6 = 256
TE = 640
NC = 2
NS = 16
CH = 128


def _sin_pe(steps, d):
    pos = np.arange(steps)[:, None].astype(np.float32)
    div = np.exp(np.arange(0, d, 2).astype(np.float32) * (-math.log(10000.0) / d))
    pe = np.zeros((steps, d), np.float32)
    pe[:, 0::2] = np.sin(pos * div)
    pe[:, 1::2] = np.cos(pos * div)
    return jnp.asarray(pe)


def _ln(x, g, b, eps=1e-5):
    mu = jnp.mean(x, axis=-1, keepdims=True)
    var = jnp.mean((x - mu) ** 2, axis=-1, keepdims=True)
    return (x - mu) * jax.lax.rsqrt(var + eps) * g + b


def _mm(x, w):
    sh = x.shape
    x2 = x.reshape(-1, sh[-1])
    y = jnp.dot(x2, w, preferred_element_type=jnp.float32)
    return y.reshape(sh[:-1] + (w.shape[-1],))


def _silu(x):
    return x * jax.nn.sigmoid(x)


def _prep_body(x_ref, mask_ref, u_ref, emb_ref, pe_ref,
               wu_ref, bu_ref, w1_ref, b1_ref, w2_ref, b2_ref,
               wm1_ref, bm1_ref, wm2_ref, bm2_ref, g_ref, bn_ref,
               h_ref):
    x = x_ref[...]
    mask = mask_ref[...]
    u = u_ref[...]
    xw = x * mask
    qu = jnp.dot(u, wu_ref[...], preferred_element_type=jnp.float32) + bu_ref[...]
    q = qu[:, None, :] + emb_ref[...][None, :, :]
    q = jnp.where(q > 0, q, 0.01 * q)
    q = jnp.maximum(_mm(q, wm1_ref[...]) + bm1_ref[...], 0.0)
    q = jnp.maximum(_mm(q, wm2_ref[...]) + bm2_ref[...], 0.0)
    q = q + pe_ref[...][:, None, :]
    h = jnp.maximum(xw * w1_ref[...] + b1_ref[...], 0.0)
    h = jnp.maximum(_mm(h, w2_ref[...]) + b2_ref[...], 0.0)
    h = jnp.where(mask > 0.5, h + q, q)
    h_ref[...] = _ln(h, g_ref[...], bn_ref[...])


def _full(shape):
    nd = len(shape)
    return pl.BlockSpec(shape, lambda i: (0,) * nd)


def _prep(xp, maskp, u, embp, pe, p):
    grid = NPAD // TN1
    ns = lambda w: pl.BlockSpec((S, TN1, w), lambda i: (0, i, 0))
    ins = [
        ns(1), ns(1), _full((S, 8)),
        pl.BlockSpec((TN1, H), lambda i: (i, 0)), _full((S, H)),
        _full((8, H)), _full((1, H)), _full((1, H)), _full((1, H)),
        _full((H, H)), _full((1, H)), _full((H, H)), _full((1, H)),
        _full((H, H)), _full((1, H)), _full((1, H)), _full((1, H)),
    ]
    args = (
        xp, maskp, u, embp, pe,
        p["u_lin"]["W"], p["u_lin"]["b"].reshape(1, H),
        p["h_enc1"]["W"].reshape(1, H), p["h_enc1"]["b"].reshape(1, H),
        p["h_enc2"]["W"], p["h_enc2"]["b"].reshape(1, H),
        p["u_mlp1"]["W"], p["u_mlp1"]["b"].reshape(1, H),
        p["u_mlp2"]["W"], p["u_mlp2"]["b"].reshape(1, H),
        p["h_norm"]["g"].reshape(1, H), p["h_norm"]["b"].reshape(1, H),
    )
    return pl.pallas_call(
        _prep_body,
        grid=(grid,),
        in_specs=ins,
        out_specs=pl.BlockSpec((S, TN1, H), lambda i: (0, i, 0)),
        out_shape=jax.ShapeDtypeStruct((S, NPAD, H), jnp.float32),
    )(*args)


def _pre_body(use_emb, use_mask,
              h_ref, x_ref, mask_ref, vemb_ref, memb_ref,
              wxs_ref, bxs_ref, ws_ref, bs_ref, wt_ref, bt_ref, gb_ref,
              hp_ref, zs_ref, zt_ref):
    h = h_ref[...]
    x = x_ref[...]
    mask = mask_ref[...]
    if use_emb:
        h = h + jnp.where(mask > 0.5, vemb_ref[...][None], memb_ref[...][None])
    h = h + (x * wxs_ref[...] + bxs_ref[...]) * mask
    hp_ref[...] = h
    zs = _mm(h, ws_ref[...]) + bs_ref[...]
    zt = _mm(h, wt_ref[...])
    gb = gb_ref[...][None]
    if use_mask:
        mb = jnp.where(mask > 0.5, 0.0, -1e9) + gb
    else:
        mb = jnp.zeros_like(mask) + gb
    pad = jnp.zeros(zs.shape[:-1] + (WID - H - 1,), jnp.float32)
    zs_ref[...] = jnp.concatenate([zs, mb, pad], axis=-1)
    zt_ref[...] = jnp.concatenate([zt, jnp.zeros_like(mb), pad], axis=-1)


def _pre(hs, xp, maskp, p, lp, layer_idx):
    grid = NPAD // TN1
    ns = lambda w: pl.BlockSpec((S, TN1, w), lambda i: (0, i, 0))
    es = pl.BlockSpec((TN1, H), lambda i: (i, 0))
    bias_total = (lp["sp_src"]["b"] + lp["sp_tgt"]["b"] + lp["sp_bias"]).reshape(1, H)
    args = (
        hs, xp, maskp, p["valid_emb_p"], p["mask_emb_p"],
        lp["x_skip"]["W"].reshape(1, H), lp["x_skip"]["b"].reshape(1, H),
        lp["sp_src"]["W"], bias_total, lp["sp_tgt"]["W"],
        jnp.zeros((1, H), jnp.float32),
        lp["sp_gate"]["b"].reshape(1, 1),
    )
    ins = [ns(H), ns(1), ns(1), es, es,
           _full((1, H)), _full((1, H)), _full((H, H)), _full((1, H)),
           _full((H, H)), _full((1, H)), _full((1, 1))]
    body = functools.partial(_pre_body, layer_idx == 1, layer_idx == 0)
    return pl.pallas_call(
        body,
        grid=(grid,),
        in_specs=ins,
        out_specs=[ns(H), ns(WID), ns(WID)],
        out_shape=[
            jax.ShapeDtypeStruct((S, NPAD, H), jnp.float32),
            jax.ShapeDtypeStruct((S, NPAD, WID), jnp.float32),
            jax.ShapeDtypeStruct((S, NPAD, WID), jnp.float32),
        ],
    )(*args)


def _edge_body(a1_ref, a2_ref, g_ref, wm_ref, p_ref):
    a = a1_ref[...] + a2_ref[...]
    feats = a[:, :, :H]
    m = _silu(feats)
    score = jnp.sum(m * g_ref[...][0, None, None, :], axis=-1, keepdims=True)
    score = score + a[:, :, H:H + 1]
    w = jnp.exp(score)
    msg = _silu(_mm(m, wm_ref[...]))
    pad = jnp.zeros((a.shape[0], a.shape[1], W2 - H - 1), jnp.float32)
    p_ref[...] = jnp.concatenate([msg * w, w, pad], axis=-1)


def _edge(A1, A2, lp):
    g = lp["sp_gate"]["W"].reshape(1, H)
    sg = A1.shape[0]
    grid = EPAD // TE
    es = pl.BlockSpec((sg, TE, WID), lambda i: (0, i, 0))
    os = pl.BlockSpec((sg, TE, W2), lambda i: (0, i, 0))
    return pl.pallas_call(
        _edge_body,
        grid=(grid,),
        in_specs=[es, es, _full((1, H)), _full((H, H))],
        out_specs=os,
        out_shape=jax.ShapeDtypeStruct((sg, EPAD, W2), jnp.float32),
    )(A1, A2, g, lp["sp_msg"]["W"])


def _sc_gather(zs_flat, zt_flat, sidx, tidx):
    nrows = sidx.shape[0]
    rows_per = nrows // (NC * NS)
    nchunk = rows_per // CH
    mesh = plsc.VectorSubcoreMesh(core_axis_name="c", subcore_axis_name="s")

    @functools.partial(
        pl.kernel, mesh=mesh,
        out_type=[
            jax.ShapeDtypeStruct((nrows, WID), jnp.float32),
            jax.ShapeDtypeStruct((nrows, WID), jnp.float32),
        ],
        scratch_types=[
            pltpu.VMEM((CH,), jnp.int32),
            pltpu.VMEM((CH,), jnp.int32),
            pltpu.VMEM((CH,), jnp.int32),
            pltpu.VMEM((CH,), jnp.int32),
            pltpu.VMEM((CH, WID), jnp.float32),
            pltpu.VMEM((CH, WID), jnp.float32),
            pltpu.VMEM((CH, WID), jnp.float32),
            pltpu.VMEM((CH, WID), jnp.float32),
            pltpu.SemaphoreType.DMA,
            pltpu.SemaphoreType.DMA,
            pltpu.SemaphoreType.DMA,
            pltpu.SemaphoreType.DMA,
            pltpu.SemaphoreType.DMA,
            pltpu.SemaphoreType.DMA,
            pltpu.SemaphoreType.DMA,
            pltpu.SemaphoreType.DMA,
        ],
    )
    def k(zs_hbm, zt_hbm, si_hbm, ti_hbm, a1_hbm, a2_hbm,
          idx1a, idx2a, idx1b, idx2b, r1a, r2a, r1b, r2b,
          sem1a, sem2a, sem1b, sem2b, ws1a, ws2a, ws1b, ws2b):
        cid = lax.axis_index("c")
        sid = lax.axis_index("s")
        wid = sid * NC + cid
        base_t = wid * rows_per

        def body(j, carry):
            ba = base_t + (2 * j) * CH
            bb = base_t + (2 * j + 1) * CH
            pltpu.sync_copy(si_hbm.at[pl.ds(ba, CH)], idx1a)
            pltpu.sync_copy(ti_hbm.at[pl.ds(ba, CH)], idx2a)
            cp1a = pltpu.async_copy(zs_hbm.at[idx1a], r1a, sem1a)
            cp2a = pltpu.async_copy(zt_hbm.at[idx2a], r2a, sem2a)
            pltpu.sync_copy(si_hbm.at[pl.ds(bb, CH)], idx1b)
            pltpu.sync_copy(ti_hbm.at[pl.ds(bb, CH)], idx2b)
            cp1b = pltpu.async_copy(zs_hbm.at[idx1b], r1b, sem1b)
            cp2b = pltpu.async_copy(zt_hbm.at[idx2b], r2b, sem2b)
            cp1a.wait()
            w1a = pltpu.async_copy(r1a, a1_hbm.at[pl.ds(ba, CH)], ws1a)
            cp2a.wait()
            w2a = pltpu.async_copy(r2a, a2_hbm.at[pl.ds(ba, CH)], ws2a)
            cp1b.wait()
            w1b = pltpu.async_copy(r1b, a1_hbm.at[pl.ds(bb, CH)], ws1b)
            cp2b.wait()
            w2b = pltpu.async_copy(r2b, a2_hbm.at[pl.ds(bb, CH)], ws2b)
            w1a.wait()
            w2a.wait()
            w1b.wait()
            w2b.wait()
            return carry

        lax.fori_loop(0, nchunk // 2, body, 0)

    return k(zs_flat, zt_flat, sidx, tidx)


def _sc_scatter(P_flat, tgt_pad, zrows):
    half = EPAD // NC
    per_tile = half // NS
    nchunk = per_tile // CH
    srows = NPAD // NS
    nst = P_flat.shape[0] // EPAD
    mesh = plsc.VectorSubcoreMesh(core_axis_name="c", subcore_axis_name="s")

    @functools.partial(
        pl.kernel, mesh=mesh,
        out_type=jax.ShapeDtypeStruct((NC, nst, NPAD, W2), jnp.float32),
        scratch_types=[
            pltpu.VMEM((CH,), jnp.int32),
            pltpu.VMEM((CH, W2), jnp.float32),
            pltpu.VMEM_SHARED((NPAD, W2), jnp.float32),
        ],
    )
    def k(p_hbm, tgt_hbm, z_hbm, out_hbm, idx1, rows, table):
        cid = lax.axis_index("c")
        sid = lax.axis_index("s")
        myslice = pl.ds(sid * srows, srows)
        pltpu.sync_copy(z_hbm, table.at[myslice])
        plsc.subcore_barrier()

        def step_body(s, carry):
            ebase0 = cid * half + sid * per_tile

            def chunk(i, c2):
                eb = ebase0 + i * CH
                pltpu.sync_copy(tgt_hbm.at[pl.ds(eb, CH)], idx1)
                pltpu.sync_copy(p_hbm.at[pl.ds(s * EPAD + eb, CH)], rows)
                pltpu.sync_copy(rows, table.at[idx1], add=True)
                return c2

            lax.fori_loop(0, nchunk, chunk, 0)
            plsc.subcore_barrier()
            pltpu.sync_copy(table.at[myslice], out_hbm.at[cid, s, myslice])
            pltpu.sync_copy(z_hbm, table.at[myslice])
            plsc.subcore_barrier()
            return carry

        lax.fori_loop(0, nst, step_body, 0)

    return k(P_flat, tgt_pad, zrows)


def _post_body(hp_ref, p1_ref, p2_ref, mask_ref,
               wts_ref, bts_ref, wtt_ref, wtm_ref, btm_ref, gt_ref, gtb_ref,
               wsk_ref, bsk_ref, gn_ref, bn_ref,
               w1_ref, b1_ref, w2_ref, b2_ref, w3_ref, b3_ref,
               h_ref, imp_ref):
    hp = hp_ref[...]
    mask = mask_ref[...]
    psum = p1_ref[...] + p2_ref[...]
    out_sp = psum[:, :, :H] / (psum[:, :, H:H + 1] + 1e-16)

    zs = _mm(hp, wts_ref[...]) + bts_ref[...]
    zt = _mm(hp, wtt_ref[...])
    pair = _silu(zs[None, :, :, :] + zt[:, None, :, :])
    score = jnp.sum(pair * gt_ref[...][0, None, None, None, :], axis=-1)
    score = score + gtb_ref[...][:, :, None]
    tmask = (mask[:, :, 0] > 0.5)[None, :, :]
    score = jnp.where(tmask, score, -1e9)
    smax = jnp.max(score, axis=1, keepdims=True)
    ex = jnp.exp(score - smax)
    alpha = ex / jnp.sum(ex, axis=1, keepdims=True)
    msg = _silu(_mm(pair, wtm_ref[...]))
    out_tm = jnp.sum(alpha[..., None] * msg, axis=1)

    hn = out_sp + out_tm + _mm(hp, wsk_ref[...]) + bsk_ref[...]
    hn = _ln(hn, gn_ref[...], bn_ref[...])
    h_ref[...] = hn
    r = jnp.maximum(_mm(hn, w1_ref[...]) + b1_ref[...], 0.0)
    r = jnp.maximum(_mm(r, w2_ref[...]) + b2_ref[...], 0.0)
    imp = jnp.sum(r * w3_ref[...][0, None, None, :], axis=-1, keepdims=True)
    imp_ref[...] = imp + b3_ref[...][None]


def _post(hp, part1, part2, maskp, lp, gate_b):
    grid = NPAD // TN6
    ns = lambda w: pl.BlockSpec((S, TN6, w), lambda i: (0, i, 0))
    bts = (lp["tm_src"]["b"] + lp["tm_tgt"]["b"] + lp["tm_bias"]).reshape(1, H)
    args = (
        hp, part1, part2, maskp,
        lp["tm_src"]["W"], bts, lp["tm_tgt"]["W"],
        lp["tm_msg"]["W"], lp["tm_msg"]["b"].reshape(1, H),
        lp["tm_gate"]["W"].reshape(1, H), lp["tm_gate"]["b"].reshape(1, 1),
        lp["skip"]["W"], lp["skip"]["b"].reshape(1, H),
        lp["norm"]["g"].reshape(1, H), lp["norm"]["b"].reshape(1, H),
        lp["ro1"]["W"], lp["ro1"]["b"].reshape(1, H),
        lp["ro2"]["W"], lp["ro2"]["b"].reshape(1, H),
        lp["ro3"]["W"].reshape(1, H), lp["ro3"]["b"].reshape(1, 1),
    )
    ins = [ns(H), ns(W2), ns(W2), ns(1),
           _full((H, H)), _full((1, H)), _full((H, H)),
           _full((H, H)), _full((1, H)), _full((1, H)), _full((1, 1)),
           _full((H, H)), _full((1, H)), _full((1, H)), _full((1, H)),
           _full((H, H)), _full((1, H)), _full((H, H)), _full((1, H)),
           _full((1, H)), _full((1, 1))]
    return pl.pallas_call(
        _post_body,
        grid=(grid,),
        in_specs=ins,
        out_specs=[ns(H), ns(1)],
        out_shape=[
            jax.ShapeDtypeStruct((S, NPAD, H), jnp.float32),
            jax.ShapeDtypeStruct((S, NPAD, 1), jnp.float32),
        ],
    )(*args)


def kernel(x, u, mask, edge_index, params):
    x3 = x[0]
    mask3 = mask[0]
    u2 = u[0]
    src = edge_index[0]
    tgt = edge_index[1]

    padn = ((0, 0), (0, NPAD - N), (0, 0))
    xp = jnp.pad(x3, padn)
    maskp = jnp.pad(mask3, padn)
    embp = jnp.pad(params["u_node_emb"], ((0, NPAD - N), (0, 0)))
    pe = _sin_pe(S, H)

    p = dict(params)
    p["valid_emb_p"] = jnp.pad(params["valid_emb"], ((0, NPAD - N), (0, 0)))
    p["mask_emb_p"] = jnp.pad(params["mask_emb"], ((0, NPAD - N), (0, 0)))

    h = _prep(xp, maskp, u2, embp, pe, p)

    src_pad = jnp.concatenate([src, jnp.zeros((EPAD - E,), jnp.int32)])
    tgt_pad = jnp.concatenate(
        [tgt, jnp.full((EPAD - E,), NPAD - 1, jnp.int32)])
    steps_off = (jnp.arange(S, dtype=jnp.int32) * NPAD)[:, None]
    sidx = (steps_off + src_pad[None, :]).reshape(-1)
    tidx = (steps_off + tgt_pad[None, :]).reshape(-1)
    zrows = jnp.zeros((NPAD // NS, W2), jnp.float32)

    imps = []
    for l, lp in enumerate(params["layers"]):
        hp, zs_ext, zt_ext = _pre(h, xp, maskp, p, lp, l)
        A1, A2 = _sc_gather(
            zs_ext.reshape(S * NPAD, WID), zt_ext.reshape(S * NPAD, WID),
            sidx, tidx)
        P = _edge(A1.reshape(S, EPAD, WID), A2.reshape(S, EPAD, WID), lp)
        parts = _sc_scatter(P.reshape(S * EPAD, W2), tgt_pad, zrows)
        h, imp = _post(hp, parts[0], parts[1], maskp, lp, None)
        imps.append(imp[:, :N, :][None])

    return (imps[1], imps[0])


# speedup vs baseline: 1.3537x; 1.0772x over previous
import functools
import math

import numpy as np
import jax
from jax import lax
import jax.numpy as jnp
from jax.experimental import pallas as pl
from jax.experimental.pallas import tpu as pltpu
from jax.experimental.pallas import tpu_sc as plsc

S = 8
N = 10000
NPAD = 10240
H = 32
E = 160000
EPAD = 163840
WID = 128
W2 = 36
TN1 = 512
TN6 = 256
TE = 640
NC = 2
NS = 16
CH = 128


def _sin_pe(steps, d):
    pos = np.arange(steps)[:, None].astype(np.float32)
    div = np.exp(np.arange(0, d, 2).astype(np.float32) * (-math.log(10000.0) / d))
    pe = np.zeros((steps, d), np.float32)
    pe[:, 0::2] = np.sin(pos * div)
    pe[:, 1::2] = np.cos(pos * div)
    return jnp.asarray(pe)


def _ln(x, g, b, eps=1e-5):
    mu = jnp.mean(x, axis=-1, keepdims=True)
    var = jnp.mean((x - mu) ** 2, axis=-1, keepdims=True)
    return (x - mu) * jax.lax.rsqrt(var + eps) * g + b


def _mm(x, w):
    sh = x.shape
    x2 = x.reshape(-1, sh[-1])
    y = jnp.dot(x2, w, preferred_element_type=jnp.float32)
    return y.reshape(sh[:-1] + (w.shape[-1],))


def _silu(x):
    return x * jax.nn.sigmoid(x)


def _prep_body(x_ref, mask_ref, u_ref, emb_ref, pe_ref,
               wu_ref, bu_ref, w1_ref, b1_ref, w2_ref, b2_ref,
               wm1_ref, bm1_ref, wm2_ref, bm2_ref, g_ref, bn_ref,
               h_ref):
    x = x_ref[...]
    mask = mask_ref[...]
    u = u_ref[...]
    xw = x * mask
    qu = jnp.dot(u, wu_ref[...], preferred_element_type=jnp.float32) + bu_ref[...]
    q = qu[:, None, :] + emb_ref[...][None, :, :]
    q = jnp.where(q > 0, q, 0.01 * q)
    q = jnp.maximum(_mm(q, wm1_ref[...]) + bm1_ref[...], 0.0)
    q = jnp.maximum(_mm(q, wm2_ref[...]) + bm2_ref[...], 0.0)
    q = q + pe_ref[...][:, None, :]
    h = jnp.maximum(xw * w1_ref[...] + b1_ref[...], 0.0)
    h = jnp.maximum(_mm(h, w2_ref[...]) + b2_ref[...], 0.0)
    h = jnp.where(mask > 0.5, h + q, q)
    h_ref[...] = _ln(h, g_ref[...], bn_ref[...])


def _full(shape):
    nd = len(shape)
    return pl.BlockSpec(shape, lambda i: (0,) * nd)


def _prep(xp, maskp, u, embp, pe, p):
    grid = NPAD // TN1
    ns = lambda w: pl.BlockSpec((S, TN1, w), lambda i: (0, i, 0))
    ins = [
        ns(1), ns(1), _full((S, 8)),
        pl.BlockSpec((TN1, H), lambda i: (i, 0)), _full((S, H)),
        _full((8, H)), _full((1, H)), _full((1, H)), _full((1, H)),
        _full((H, H)), _full((1, H)), _full((H, H)), _full((1, H)),
        _full((H, H)), _full((1, H)), _full((1, H)), _full((1, H)),
    ]
    args = (
        xp, maskp, u, embp, pe,
        p["u_lin"]["W"], p["u_lin"]["b"].reshape(1, H),
        p["h_enc1"]["W"].reshape(1, H), p["h_enc1"]["b"].reshape(1, H),
        p["h_enc2"]["W"], p["h_enc2"]["b"].reshape(1, H),
        p["u_mlp1"]["W"], p["u_mlp1"]["b"].reshape(1, H),
        p["u_mlp2"]["W"], p["u_mlp2"]["b"].reshape(1, H),
        p["h_norm"]["g"].reshape(1, H), p["h_norm"]["b"].reshape(1, H),
    )
    return pl.pallas_call(
        _prep_body,
        grid=(grid,),
        in_specs=ins,
        out_specs=pl.BlockSpec((S, TN1, H), lambda i: (0, i, 0)),
        out_shape=jax.ShapeDtypeStruct((S, NPAD, H), jnp.float32),
    )(*args)


def _pre_body(use_emb, use_mask,
              h_ref, x_ref, mask_ref, vemb_ref, memb_ref,
              wxs_ref, bxs_ref, ws_ref, bs_ref, wt_ref, bt_ref, gb_ref,
              hp_ref, zs_ref, zt_ref):
    h = h_ref[...]
    x = x_ref[...]
    mask = mask_ref[...]
    if use_emb:
        h = h + jnp.where(mask > 0.5, vemb_ref[...][None], memb_ref[...][None])
    h = h + (x * wxs_ref[...] + bxs_ref[...]) * mask
    hp_ref[...] = h
    zs = _mm(h, ws_ref[...]) + bs_ref[...]
    zt = _mm(h, wt_ref[...])
    gb = gb_ref[...][None]
    if use_mask:
        mb = jnp.where(mask > 0.5, 0.0, -1e9) + gb
    else:
        mb = jnp.zeros_like(mask) + gb
    pad = jnp.zeros(zs.shape[:-1] + (WID - H - 1,), jnp.float32)
    zs_ref[...] = jnp.concatenate([zs, mb, pad], axis=-1)
    zt_ref[...] = jnp.concatenate([zt, jnp.zeros_like(mb), pad], axis=-1)


def _pre(hs, xp, maskp, p, lp, layer_idx):
    grid = NPAD // TN1
    ns = lambda w: pl.BlockSpec((S, TN1, w), lambda i: (0, i, 0))
    es = pl.BlockSpec((TN1, H), lambda i: (i, 0))
    bias_total = (lp["sp_src"]["b"] + lp["sp_tgt"]["b"] + lp["sp_bias"]).reshape(1, H)
    args = (
        hs, xp, maskp, p["valid_emb_p"], p["mask_emb_p"],
        lp["x_skip"]["W"].reshape(1, H), lp["x_skip"]["b"].reshape(1, H),
        lp["sp_src"]["W"], bias_total, lp["sp_tgt"]["W"],
        jnp.zeros((1, H), jnp.float32),
        lp["sp_gate"]["b"].reshape(1, 1),
    )
    ins = [ns(H), ns(1), ns(1), es, es,
           _full((1, H)), _full((1, H)), _full((H, H)), _full((1, H)),
           _full((H, H)), _full((1, H)), _full((1, 1))]
    body = functools.partial(_pre_body, layer_idx == 1, layer_idx == 0)
    return pl.pallas_call(
        body,
        grid=(grid,),
        in_specs=ins,
        out_specs=[ns(H), ns(WID), ns(WID)],
        out_shape=[
            jax.ShapeDtypeStruct((S, NPAD, H), jnp.float32),
            jax.ShapeDtypeStruct((S, NPAD, WID), jnp.float32),
            jax.ShapeDtypeStruct((S, NPAD, WID), jnp.float32),
        ],
    )(*args)


def _edge_body(a1_ref, a2_ref, g_ref, wm_ref, p_ref):
    a = a1_ref[...] + a2_ref[...]
    feats = a[:, :, :H]
    m = _silu(feats)
    score = jnp.sum(m * g_ref[...][0, None, None, :], axis=-1, keepdims=True)
    score = score + a[:, :, H:H + 1]
    w = jnp.exp(score)
    msg = _silu(_mm(m, wm_ref[...]))
    pad = jnp.zeros((a.shape[0], a.shape[1], W2 - H - 1), jnp.float32)
    p_ref[...] = jnp.concatenate([msg * w, w, pad], axis=-1)


def _edge(A1, A2, lp):
    g = lp["sp_gate"]["W"].reshape(1, H)
    sg = A1.shape[0]
    grid = EPAD // TE
    es = pl.BlockSpec((sg, TE, WID), lambda i: (0, i, 0))
    os = pl.BlockSpec((sg, TE, W2), lambda i: (0, i, 0))
    return pl.pallas_call(
        _edge_body,
        grid=(grid,),
        in_specs=[es, es, _full((1, H)), _full((H, H))],
        out_specs=os,
        out_shape=jax.ShapeDtypeStruct((sg, EPAD, W2), jnp.float32),
    )(A1, A2, g, lp["sp_msg"]["W"])


def _sc_gather(zs_flat, zt_flat, sidx, tidx):
    nrows = sidx.shape[0]
    rows_per = nrows // (NC * NS)
    nchunk = rows_per // CH
    mesh = plsc.VectorSubcoreMesh(core_axis_name="c", subcore_axis_name="s")

    @functools.partial(
        pl.kernel, mesh=mesh,
        out_type=[
            jax.ShapeDtypeStruct((nrows, WID), jnp.float32),
            jax.ShapeDtypeStruct((nrows, WID), jnp.float32),
        ],
        scratch_types=[
            pltpu.VMEM((CH,), jnp.int32),
            pltpu.VMEM((CH,), jnp.int32),
            pltpu.VMEM((CH,), jnp.int32),
            pltpu.VMEM((CH,), jnp.int32),
            pltpu.VMEM((CH, WID), jnp.float32),
            pltpu.VMEM((CH, WID), jnp.float32),
            pltpu.VMEM((CH, WID), jnp.float32),
            pltpu.VMEM((CH, WID), jnp.float32),
            pltpu.SemaphoreType.DMA,
            pltpu.SemaphoreType.DMA,
            pltpu.SemaphoreType.DMA,
            pltpu.SemaphoreType.DMA,
            pltpu.SemaphoreType.DMA,
            pltpu.SemaphoreType.DMA,
            pltpu.SemaphoreType.DMA,
            pltpu.SemaphoreType.DMA,
        ],
    )
    def k(zs_hbm, zt_hbm, si_hbm, ti_hbm, a1_hbm, a2_hbm,
          idx1a, idx2a, idx1b, idx2b, r1a, r2a, r1b, r2b,
          sem1a, sem2a, sem1b, sem2b, ws1a, ws2a, ws1b, ws2b):
        cid = lax.axis_index("c")
        sid = lax.axis_index("s")
        wid = sid * NC + cid
        base_t = wid * rows_per

        def body(j, carry):
            ba = base_t + (2 * j) * CH
            bb = base_t + (2 * j + 1) * CH
            pltpu.sync_copy(si_hbm.at[pl.ds(ba, CH)], idx1a)
            pltpu.sync_copy(ti_hbm.at[pl.ds(ba, CH)], idx2a)
            cp1a = pltpu.async_copy(zs_hbm.at[idx1a], r1a, sem1a)
            cp2a = pltpu.async_copy(zt_hbm.at[idx2a], r2a, sem2a)
            pltpu.sync_copy(si_hbm.at[pl.ds(bb, CH)], idx1b)
            pltpu.sync_copy(ti_hbm.at[pl.ds(bb, CH)], idx2b)
            cp1b = pltpu.async_copy(zs_hbm.at[idx1b], r1b, sem1b)
            cp2b = pltpu.async_copy(zt_hbm.at[idx2b], r2b, sem2b)
            cp1a.wait()
            w1a = pltpu.async_copy(r1a, a1_hbm.at[pl.ds(ba, CH)], ws1a)
            cp2a.wait()
            w2a = pltpu.async_copy(r2a, a2_hbm.at[pl.ds(ba, CH)], ws2a)
            cp1b.wait()
            w1b = pltpu.async_copy(r1b, a1_hbm.at[pl.ds(bb, CH)], ws1b)
            cp2b.wait()
            w2b = pltpu.async_copy(r2b, a2_hbm.at[pl.ds(bb, CH)], ws2b)
            w1a.wait()
            w2a.wait()
            w1b.wait()
            w2b.wait()
            return carry

        lax.fori_loop(0, nchunk // 2, body, 0)

    return k(zs_flat, zt_flat, sidx, tidx)


def _sc_scatter(P_flat, tgt_pad, zrows):
    half = EPAD // NC
    per_tile = half // NS
    nchunk = per_tile // CH
    srows = NPAD // NS
    nst = P_flat.shape[0] // EPAD
    mesh = plsc.VectorSubcoreMesh(core_axis_name="c", subcore_axis_name="s")

    @functools.partial(
        pl.kernel, mesh=mesh,
        out_type=jax.ShapeDtypeStruct((NC, nst, NPAD, W2), jnp.float32),
        scratch_types=[
            pltpu.VMEM((CH,), jnp.int32),
            pltpu.VMEM((CH,), jnp.int32),
            pltpu.VMEM((CH, W2), jnp.float32),
            pltpu.VMEM((CH, W2), jnp.float32),
            pltpu.VMEM_SHARED((NPAD, W2), jnp.float32),
            pltpu.SemaphoreType.DMA,
            pltpu.SemaphoreType.DMA,
            pltpu.SemaphoreType.DMA,
            pltpu.SemaphoreType.DMA,
        ],
    )
    def k(p_hbm, tgt_hbm, z_hbm, out_hbm, idxa, idxb, rowsa, rowsb,
          table, sia, sib, sra, srb):
        cid = lax.axis_index("c")
        sid = lax.axis_index("s")
        myslice = pl.ds(sid * srows, srows)
        pltpu.sync_copy(z_hbm, table.at[myslice])
        plsc.subcore_barrier()

        def step_body(s, carry):
            ebase0 = cid * half + sid * per_tile

            def chunk(j, c2):
                eba = ebase0 + (2 * j) * CH
                ebb = ebase0 + (2 * j + 1) * CH
                cia = pltpu.async_copy(tgt_hbm.at[pl.ds(eba, CH)], idxa, sia)
                cra = pltpu.async_copy(
                    p_hbm.at[pl.ds(s * EPAD + eba, CH)], rowsa, sra)
                cib = pltpu.async_copy(tgt_hbm.at[pl.ds(ebb, CH)], idxb, sib)
                crb = pltpu.async_copy(
                    p_hbm.at[pl.ds(s * EPAD + ebb, CH)], rowsb, srb)
                cia.wait()
                cra.wait()
                pltpu.sync_copy(rowsa, table.at[idxa], add=True)
                cib.wait()
                crb.wait()
                pltpu.sync_copy(rowsb, table.at[idxb], add=True)
                return c2

            lax.fori_loop(0, nchunk // 2, chunk, 0)
            plsc.subcore_barrier()
            pltpu.sync_copy(table.at[myslice], out_hbm.at[cid, s, myslice])
            pltpu.sync_copy(z_hbm, table.at[myslice])
            plsc.subcore_barrier()
            return carry

        lax.fori_loop(0, nst, step_body, 0)

    return k(P_flat, tgt_pad, zrows)


def _post_body(hp_ref, p1_ref, p2_ref, mask_ref,
               wts_ref, bts_ref, wtt_ref, wtm_ref, btm_ref, gt_ref, gtb_ref,
               wsk_ref, bsk_ref, gn_ref, bn_ref,
               w1_ref, b1_ref, w2_ref, b2_ref, w3_ref, b3_ref,
               h_ref, imp_ref):
    hp = hp_ref[...]
    mask = mask_ref[...]
    psum = p1_ref[...] + p2_ref[...]
    out_sp = psum[:, :, :H] / (psum[:, :, H:H + 1] + 1e-16)

    zs = _mm(hp, wts_ref[...]) + bts_ref[...]
    zt = _mm(hp, wtt_ref[...])
    pair = _silu(zs[None, :, :, :] + zt[:, None, :, :])
    score = jnp.sum(pair * gt_ref[...][0, None, None, None, :], axis=-1)
    score = score + gtb_ref[...][:, :, None]
    tmask = (mask[:, :, 0] > 0.5)[None, :, :]
    score = jnp.where(tmask, score, -1e9)
    smax = jnp.max(score, axis=1, keepdims=True)
    ex = jnp.exp(score - smax)
    alpha = ex / jnp.sum(ex, axis=1, keepdims=True)
    msg = _silu(_mm(pair, wtm_ref[...]))
    out_tm = jnp.sum(alpha[..., None] * msg, axis=1)

    hn = out_sp + out_tm + _mm(hp, wsk_ref[...]) + bsk_ref[...]
    hn = _ln(hn, gn_ref[...], bn_ref[...])
    h_ref[...] = hn
    r = jnp.maximum(_mm(hn, w1_ref[...]) + b1_ref[...], 0.0)
    r = jnp.maximum(_mm(r, w2_ref[...]) + b2_ref[...], 0.0)
    imp = jnp.sum(r * w3_ref[...][0, None, None, :], axis=-1, keepdims=True)
    imp_ref[...] = imp + b3_ref[...][None]


def _post(hp, part1, part2, maskp, lp, gate_b):
    grid = NPAD // TN6
    ns = lambda w: pl.BlockSpec((S, TN6, w), lambda i: (0, i, 0))
    bts = (lp["tm_src"]["b"] + lp["tm_tgt"]["b"] + lp["tm_bias"]).reshape(1, H)
    args = (
        hp, part1, part2, maskp,
        lp["tm_src"]["W"], bts, lp["tm_tgt"]["W"],
        lp["tm_msg"]["W"], lp["tm_msg"]["b"].reshape(1, H),
        lp["tm_gate"]["W"].reshape(1, H), lp["tm_gate"]["b"].reshape(1, 1),
        lp["skip"]["W"], lp["skip"]["b"].reshape(1, H),
        lp["norm"]["g"].reshape(1, H), lp["norm"]["b"].reshape(1, H),
        lp["ro1"]["W"], lp["ro1"]["b"].reshape(1, H),
        lp["ro2"]["W"], lp["ro2"]["b"].reshape(1, H),
        lp["ro3"]["W"].reshape(1, H), lp["ro3"]["b"].reshape(1, 1),
    )
    ins = [ns(H), ns(W2), ns(W2), ns(1),
           _full((H, H)), _full((1, H)), _full((H, H)),
           _full((H, H)), _full((1, H)), _full((1, H)), _full((1, 1)),
           _full((H, H)), _full((1, H)), _full((1, H)), _full((1, H)),
           _full((H, H)), _full((1, H)), _full((H, H)), _full((1, H)),
           _full((1, H)), _full((1, 1))]
    return pl.pallas_call(
        _post_body,
        grid=(grid,),
        in_specs=ins,
        out_specs=[ns(H), ns(1)],
        out_shape=[
            jax.ShapeDtypeStruct((S, NPAD, H), jnp.float32),
            jax.ShapeDtypeStruct((S, NPAD, 1), jnp.float32),
        ],
    )(*args)


def kernel(x, u, mask, edge_index, params):
    x3 = x[0]
    mask3 = mask[0]
    u2 = u[0]
    src = edge_index[0]
    tgt = edge_index[1]

    padn = ((0, 0), (0, NPAD - N), (0, 0))
    xp = jnp.pad(x3, padn)
    maskp = jnp.pad(mask3, padn)
    embp = jnp.pad(params["u_node_emb"], ((0, NPAD - N), (0, 0)))
    pe = _sin_pe(S, H)

    p = dict(params)
    p["valid_emb_p"] = jnp.pad(params["valid_emb"], ((0, NPAD - N), (0, 0)))
    p["mask_emb_p"] = jnp.pad(params["mask_emb"], ((0, NPAD - N), (0, 0)))

    h = _prep(xp, maskp, u2, embp, pe, p)

    src_pad = jnp.concatenate([src, jnp.zeros((EPAD - E,), jnp.int32)])
    tgt_pad = jnp.concatenate(
        [tgt, jnp.full((EPAD - E,), NPAD - 1, jnp.int32)])
    steps_off = (jnp.arange(S, dtype=jnp.int32) * NPAD)[:, None]
    sidx = (steps_off + src_pad[None, :]).reshape(-1)
    tidx = (steps_off + tgt_pad[None, :]).reshape(-1)
    zrows = jnp.zeros((NPAD // NS, W2), jnp.float32)

    imps = []
    for l, lp in enumerate(params["layers"]):
        hp, zs_ext, zt_ext = _pre(h, xp, maskp, p, lp, l)
        A1, A2 = _sc_gather(
            zs_ext.reshape(S * NPAD, WID), zt_ext.reshape(S * NPAD, WID),
            sidx, tidx)
        P = _edge(A1.reshape(S, EPAD, WID), A2.reshape(S, EPAD, WID), lp)
        parts = _sc_scatter(P.reshape(S * EPAD, W2), tgt_pad, zrows)
        h, imp = _post(hp, parts[0], parts[1], maskp, lp, None)
        imps.append(imp[:, :N, :][None])

    return (imps[1], imps[0])
